# baseline (speedup 1.0000x reference)
import jax
import jax.numpy as jnp
from jax.experimental import pallas as pl

K = 20
SLOPE = 0.2


def _knn(x, k):
    inner = -2.0 * jnp.matmul(jnp.transpose(x, (0, 2, 1)), x)
    xx = jnp.sum(x ** 2, axis=1, keepdims=True)
    pairwise_distance = -xx - inner - jnp.transpose(xx, (0, 2, 1))
    idx = jax.lax.top_k(pairwise_distance, k)[1]
    return idx


def _get_graph_feature(x, k):
    B, C, N = x.shape
    idx = _knn(x, k)
    xt = jnp.transpose(x, (0, 2, 1))
    feature = jax.vmap(lambda pts, id_: pts[id_])(xt, idx)
    xr = jnp.broadcast_to(xt[:, :, None, :], (B, N, k, C))
    feat = jnp.concatenate([feature - xr, xr], axis=3)
    return jnp.transpose(feat, (0, 3, 1, 2))


def _leaky(x, s):
    return jnp.where(x >= 0, x, s * x)


def _gcm_block(x, W, k, slope):
    f = _get_graph_feature(x, k)
    y = jnp.einsum('oc,bcnk->bonk', W, f)
    y = _leaky(y, slope)
    return jnp.max(y, axis=-1)


def _final_kernel(g_ref, out_ref):
    g = g_ref[...]
    g = jnp.where(g >= 0, g, SLOPE * g)
    out_ref[...] = jnp.max(g, axis=-1)


def kernel(x, W1, W2, W3, W4, Wg):
    h1 = _gcm_block(x, W1, K, SLOPE)
    h2 = _gcm_block(h1, W2, K, SLOPE)
    h3 = _gcm_block(h2, W3, K, SLOPE)
    h4 = _gcm_block(h3, W4, K, SLOPE)
    x_cat = jnp.concatenate([h1, h2, h3, h4], axis=1)
    g = jnp.einsum('oc,bcn->bon', Wg, x_cat)
    out = pl.pallas_call(
        _final_kernel,
        out_shape=jax.ShapeDtypeStruct((g.shape[0], g.shape[1]), g.dtype),
    )(g)
    return out


# speedup vs baseline: 10.4409x; 10.4409x over previous
import functools

import jax
import jax.numpy as jnp
from jax import lax
from jax.experimental import pallas as pl
from jax.experimental.pallas import tpu as pltpu
from jax.experimental.pallas import tpu_sc as plsc

K = 20
SLOPE = 0.2
DP = 128


def _knn_body(xt_ref, xf_ref, idx_ref, d_ref, *, TN, N):
    xt = xt_ref[0]
    xf = xf_ref[0]
    g = lax.dot_general(xt, xf, (((1,), (0,)), ((), ())),
                        preferred_element_type=jnp.float32)
    xxt = jnp.sum(xt * xt, axis=1, keepdims=True)
    xxf = jnp.sum(xf * xf, axis=0, keepdims=True)
    d_ref[...] = 2.0 * g - xxt - xxf
    col = lax.broadcasted_iota(jnp.int32, (TN, N), 1)
    rows = []
    for i in range(K):
        d = d_ref[...]
        rm = jnp.max(d, axis=1, keepdims=True)
        cand = jnp.where(d == rm, col, N)
        am = jnp.min(cand, axis=1)
        rows.append(am)
        d_ref[...] = jnp.where(col == am[:, None], -jnp.inf, d)
    idx_ref[0] = jnp.stack(rows, axis=1)


def _uv_body(xt_ref, xf_ref, wu_ref, wv_ref, idx_ref, u_ref, v_ref, d_ref,
             *, TN, N):
    _knn_body(xt_ref, xf_ref, idx_ref, d_ref, TN=TN, N=N)
    xt = xt_ref[0]
    u_ref[0] = lax.dot_general(xt, wu_ref[...], (((1,), (0,)), ((), ())),
                               preferred_element_type=jnp.float32)
    v_ref[0] = lax.dot_general(xt, wv_ref[...], (((1,), (0,)), ((), ())),
                               preferred_element_type=jnp.float32)


def _knn_tc(xt, xf, TN=256):
    B, N, C = xt.shape
    body = functools.partial(_knn_body, TN=TN, N=N)
    return pl.pallas_call(
        body,
        grid=(B, N // TN),
        in_specs=[
            pl.BlockSpec((1, TN, C), lambda b, j: (b, j, 0)),
            pl.BlockSpec((1, C, N), lambda b, j: (b, 0, 0)),
        ],
        out_specs=pl.BlockSpec((1, TN, K), lambda b, j: (b, j, 0)),
        out_shape=jax.ShapeDtypeStruct((B, N, K), jnp.int32),
        scratch_shapes=[pltpu.VMEM((TN, N), jnp.float32)],
    )(xt, xf)


def _knn_uv_tc(xt, xf, wut, wvt, TN=256):
    B, N, C = xt.shape
    OUTP = wut.shape[1]
    OUT = wvt.shape[1]
    body = functools.partial(_uv_body, TN=TN, N=N)
    return pl.pallas_call(
        body,
        grid=(B, N // TN),
        in_specs=[
            pl.BlockSpec((1, TN, C), lambda b, j: (b, j, 0)),
            pl.BlockSpec((1, C, N), lambda b, j: (b, 0, 0)),
            pl.BlockSpec((C, OUTP), lambda b, j: (0, 0)),
            pl.BlockSpec((C, OUT), lambda b, j: (0, 0)),
        ],
        out_specs=[
            pl.BlockSpec((1, TN, K), lambda b, j: (b, j, 0)),
            pl.BlockSpec((1, TN, OUTP), lambda b, j: (b, j, 0)),
            pl.BlockSpec((1, TN, OUT), lambda b, j: (b, j, 0)),
        ],
        out_shape=[
            jax.ShapeDtypeStruct((B, N, K), jnp.int32),
            jax.ShapeDtypeStruct((B, N, OUTP), jnp.float32),
            jax.ShapeDtypeStruct((B, N, OUT), jnp.float32),
        ],
        scratch_shapes=[pltpu.VMEM((TN, N), jnp.float32)],
    )(xt, xf, wut, wvt)


def _sc_gather_rows(tab, ifl, G=8):
    B, N, D = tab.shape
    info = plsc.get_sparse_core_info()
    NC = info.num_cores
    NW = NC * info.num_subcores
    WPB = NW // B
    NP = N // WPB
    mesh = plsc.VectorSubcoreMesh(core_axis_name="c", subcore_axis_name="s")

    @functools.partial(
        pl.kernel, mesh=mesh,
        out_type=jax.ShapeDtypeStruct((B, N * K, D), jnp.float32),
        scratch_types=[
            pltpu.VMEM((NP * K,), jnp.int32),
            pltpu.VMEM((G * K, D), jnp.float32),
            pltpu.SemaphoreType.DMA,
        ],
    )
    def k(t_hbm, i_hbm, o_hbm, idx_v, rows2, sem):
        wid = lax.axis_index("s") * NC + lax.axis_index("c")
        b = wid // WPB
        p0 = (wid % WPB) * NP
        pltpu.sync_copy(i_hbm.at[b, pl.ds(p0 * K, NP * K)], idx_v)

        def g_body(g, carry):
            pltpu.async_copy(
                t_hbm.at[b].at[idx_v.at[pl.ds(g * (G * K), G * K)]],
                rows2, sem).wait()
            pltpu.sync_copy(
                rows2, o_hbm.at[b, pl.ds((p0 + g * G) * K, G * K), :])
            return carry

        lax.fori_loop(0, NP // G, g_body, 0)

    return k(tab, ifl)


def _conv_body(xj_ref, xt_ref, w_ref, h_ref, *, TN, C, CP2, OUT):
    xj = xj_ref[0][:, :C]
    xn = xt_ref[0]
    xnk = jnp.broadcast_to(xn[:, None, :], (TN, K, C)).reshape(TN * K, C)
    a = (xj - xnk).astype(jnp.bfloat16)
    bq = xnk.astype(jnp.bfloat16)
    parts = [a, bq]
    if CP2 > 2 * C:
        parts.append(jnp.zeros((TN * K, CP2 - 2 * C), jnp.bfloat16))
    feat = jnp.concatenate(parts, axis=1)
    y = lax.dot_general(feat, w_ref[...], (((1,), (0,)), ((), ())),
                        preferred_element_type=jnp.float32)
    y = y.reshape(TN, K, OUT)
    h = jnp.max(y, axis=1)
    h_ref[0] = jnp.where(h >= 0, h, SLOPE * h)


def _conv_tc(xj, xt, w16, TN=128):
    B, N, C = xt.shape
    CP2, OUT = w16.shape
    body = functools.partial(_conv_body, TN=TN, C=C, CP2=CP2, OUT=OUT)
    return pl.pallas_call(
        body,
        grid=(B, N // TN),
        in_specs=[
            pl.BlockSpec((1, TN * K, DP), lambda b, j: (b, j, 0)),
            pl.BlockSpec((1, TN, C), lambda b, j: (b, j, 0)),
            pl.BlockSpec((CP2, OUT), lambda b, j: (0, 0)),
        ],
        out_specs=pl.BlockSpec((1, TN, OUT), lambda b, j: (b, j, 0)),
        out_shape=jax.ShapeDtypeStruct((B, N, OUT), jnp.float32),
    )(xj, xt, w16)


def _sc_gather_max(ut, vfl, ifl, OUT, G=8):
    B, N, OUTP = ut.shape
    info = plsc.get_sparse_core_info()
    NC = info.num_cores
    NW = NC * info.num_subcores
    WPB = NW // B
    NP = N // WPB
    mesh = plsc.VectorSubcoreMesh(core_axis_name="c", subcore_axis_name="s")

    @functools.partial(
        pl.kernel, mesh=mesh,
        out_type=jax.ShapeDtypeStruct((B, N * OUT), jnp.float32),
        scratch_types=[
            pltpu.VMEM((NP * K,), jnp.int32),
            pltpu.VMEM((G * K, OUTP), jnp.float32),
            pltpu.VMEM((G * OUT,), jnp.float32),
            pltpu.VMEM((G * OUT,), jnp.float32),
            pltpu.SemaphoreType.DMA,
        ],
    )
    def k(u_hbm, v_hbm, i_hbm, h_hbm, idx_v, rows2, vg, hg, sem):
        wid = lax.axis_index("s") * NC + lax.axis_index("c")
        b = wid // WPB
        p0 = (wid % WPB) * NP
        pltpu.sync_copy(i_hbm.at[b, pl.ds(p0 * K, NP * K)], idx_v)

        def g_body(g, carry):
            pltpu.async_copy(
                u_hbm.at[b].at[idx_v.at[pl.ds(g * (G * K), G * K)]],
                rows2, sem).wait()
            pltpu.sync_copy(
                v_hbm.at[b, pl.ds((p0 + g * G) * OUT, G * OUT)], vg)

            def c_body(c, carry2):
                co = c * 16
                for p in range(G):
                    m = jnp.full((16,), -jnp.inf, jnp.float32)
                    for kk in range(K):
                        m = jnp.maximum(m, rows2[p * K + kk, pl.ds(co, 16)])
                    t = m + vg[pl.ds(p * OUT + co, 16)]
                    hg[pl.ds(p * OUT + co, 16)] = (
                        jnp.where(t >= 0, t, SLOPE * t))
                return carry2

            lax.fori_loop(0, OUT // 16, c_body, 0)
            pltpu.sync_copy(
                hg, h_hbm.at[b, pl.ds((p0 + g * G) * OUT, G * OUT)])
            return carry

        lax.fori_loop(0, NP // G, g_body, 0)

    return k(ut, vfl, ifl)


def _final_body(h1_ref, h2_ref, h3_ref, h4_ref, w1_ref, w2_ref, w3_ref,
                w4_ref, out_ref, *, NT):
    j = pl.program_id(1)
    dn = (((1,), (0,)), ((), ()))
    g = lax.dot_general(w1_ref[...], h1_ref[0], dn,
                        preferred_element_type=jnp.float32)
    g += lax.dot_general(w2_ref[...], h2_ref[0], dn,
                         preferred_element_type=jnp.float32)
    g += lax.dot_general(w3_ref[...], h3_ref[0], dn,
                         preferred_element_type=jnp.float32)
    g += lax.dot_general(w4_ref[...], h4_ref[0], dn,
                         preferred_element_type=jnp.float32)
    part = jnp.max(g, axis=1)[None, None, :]

    @pl.when(j == 0)
    def _():
        out_ref[...] = jnp.full(out_ref.shape, -jnp.inf, jnp.float32)

    acc = jnp.maximum(out_ref[...], part)
    last = j == NT - 1
    out_ref[...] = jnp.where(last, jnp.where(acc >= 0, acc, SLOPE * acc), acc)


def _final(h1, h2, h3, h4, Wg, TND=512):
    B, _, N = h1.shape
    NT = N // TND
    O = Wg.shape[0]
    c1, c2, c3 = h1.shape[1], h2.shape[1], h3.shape[1]
    w1 = Wg[:, :c1]
    w2 = Wg[:, c1:c1 + c2]
    w3 = Wg[:, c1 + c2:c1 + c2 + c3]
    w4 = Wg[:, c1 + c2 + c3:]
    body = functools.partial(_final_body, NT=NT)

    def hspec(C):
        return pl.BlockSpec((1, C, TND), lambda b, j: (b, 0, j))

    def wspec(C):
        return pl.BlockSpec((O, C), lambda b, j: (0, 0))

    return pl.pallas_call(
        body,
        grid=(B, NT),
        in_specs=[hspec(c1), hspec(c2), hspec(c3), hspec(h4.shape[1]),
                  wspec(c1), wspec(c2), wspec(c3), wspec(w4.shape[1])],
        out_specs=pl.BlockSpec((1, 1, O), lambda b, j: (b, 0, 0)),
        out_shape=jax.ShapeDtypeStruct((B, 1, O), jnp.float32),
    )(h1, h2, h3, h4, w1, w2, w3, w4).reshape(B, O)


def _exact_block(xf_knn, xt_knn, xt, W):
    B, N, C = xt.shape
    OUT, C2 = W.shape
    CP2 = max(2 * C, 8)
    w16 = jnp.pad(W.astype(jnp.bfloat16).T, ((0, CP2 - C2), (0, 0)))
    idx = _knn_tc(xt_knn, xf_knn)
    tab = jnp.pad(xt, ((0, 0), (0, 0), (0, DP - C)))
    xj = _sc_gather_rows(tab, idx.reshape(B, N * K))
    return _conv_tc(xj, xt, w16)


def _fast_block(xf, xt, W):
    B, N, C = xt.shape
    OUT = W.shape[0]
    OUTP = max(OUT, 128)
    wa = W[:, :C].T
    wv = (W[:, C:] - W[:, :C]).T
    wut = jnp.pad(wa, ((0, 0), (0, OUTP - OUT)))
    idx, ut, vt = _knn_uv_tc(xt, xf, wut, wv)
    hfl = _sc_gather_max(ut, vt.reshape(B, N * OUT), idx.reshape(B, N * K),
                         OUT)
    return hfl.reshape(B, N, OUT)


def kernel(x, W1, W2, W3, W4, Wg):
    B, C0, N = x.shape
    xp = jnp.pad(x, ((0, 0), (0, 8 - C0), (0, 0)))
    xtp = jnp.transpose(xp, (0, 2, 1))
    h1t = _exact_block(xp, xtp, xtp[:, :, :C0], W1)
    h1 = jnp.transpose(h1t, (0, 2, 1))
    h2t = _exact_block(h1, h1t, h1t, W2)
    h2 = jnp.transpose(h2t, (0, 2, 1))
    h3t = _exact_block(h2, h2t, h2t, W3)
    h3 = jnp.transpose(h3t, (0, 2, 1))
    h4t = _fast_block(h3, h3t, W4)
    h4 = jnp.transpose(h4t, (0, 2, 1))
    return _final(h1, h2, h3, h4, Wg)


# speedup vs baseline: 13.1765x; 1.2620x over previous
import functools

import jax
import jax.numpy as jnp
from jax import lax
from jax.experimental import pallas as pl
from jax.experimental.pallas import tpu as pltpu
from jax.experimental.pallas import tpu_sc as plsc

K = 20
SLOPE = 0.2
DP = 128


def _knn_body(xt_ref, xf_ref, idx_ref, d_ref, *, TN, N):
    xt = xt_ref[0]
    xf = xf_ref[0]
    g = lax.dot_general(xt, xf, (((1,), (0,)), ((), ())),
                        preferred_element_type=jnp.float32)
    xxt = jnp.sum(xt * xt, axis=1, keepdims=True)
    xxf = jnp.sum(xf * xf, axis=0, keepdims=True)
    d_ref[...] = 2.0 * g - xxt - xxf
    colf = lax.broadcasted_iota(jnp.int32, (TN, N), 1).astype(jnp.float32)
    rows = []
    for i in range(K):
        d = d_ref[...]
        rm = jnp.max(d, axis=1, keepdims=True)
        cand = jnp.where(d == rm, colf, jnp.float32(N))
        am = jnp.min(cand, axis=1, keepdims=True)
        rows.append(am.astype(jnp.int32))
        d_ref[...] = jnp.where(cand == am, -jnp.inf, d)
    idx_ref[0] = jnp.concatenate(rows, axis=1)


def _uv_body(xt_ref, xf_ref, wu_ref, wv_ref, idx_ref, u_ref, v_ref, d_ref,
             *, TN, N):
    _knn_body(xt_ref, xf_ref, idx_ref, d_ref, TN=TN, N=N)
    xt = xt_ref[0]
    u_ref[0] = lax.dot_general(xt, wu_ref[...], (((1,), (0,)), ((), ())),
                               preferred_element_type=jnp.float32)
    v_ref[0] = lax.dot_general(xt, wv_ref[...], (((1,), (0,)), ((), ())),
                               preferred_element_type=jnp.float32)


def _knn_tc(xt, xf, TN=256):
    B, N, C = xt.shape
    body = functools.partial(_knn_body, TN=TN, N=N)
    return pl.pallas_call(
        body,
        grid=(B, N // TN),
        in_specs=[
            pl.BlockSpec((1, TN, C), lambda b, j: (b, j, 0)),
            pl.BlockSpec((1, C, N), lambda b, j: (b, 0, 0)),
        ],
        out_specs=pl.BlockSpec((1, TN, K), lambda b, j: (b, j, 0)),
        out_shape=jax.ShapeDtypeStruct((B, N, K), jnp.int32),
        scratch_shapes=[pltpu.VMEM((TN, N), jnp.float32)],
    )(xt, xf)


def _knn_uv_tc(xt, xf, wut, wvt, TN=256):
    B, N, C = xt.shape
    OUTP = wut.shape[1]
    OUT = wvt.shape[1]
    body = functools.partial(_uv_body, TN=TN, N=N)
    return pl.pallas_call(
        body,
        grid=(B, N // TN),
        in_specs=[
            pl.BlockSpec((1, TN, C), lambda b, j: (b, j, 0)),
            pl.BlockSpec((1, C, N), lambda b, j: (b, 0, 0)),
            pl.BlockSpec((C, OUTP), lambda b, j: (0, 0)),
            pl.BlockSpec((C, OUT), lambda b, j: (0, 0)),
        ],
        out_specs=[
            pl.BlockSpec((1, TN, K), lambda b, j: (b, j, 0)),
            pl.BlockSpec((1, TN, OUTP), lambda b, j: (b, j, 0)),
            pl.BlockSpec((1, TN, OUT), lambda b, j: (b, j, 0)),
        ],
        out_shape=[
            jax.ShapeDtypeStruct((B, N, K), jnp.int32),
            jax.ShapeDtypeStruct((B, N, OUTP), jnp.float32),
            jax.ShapeDtypeStruct((B, N, OUT), jnp.float32),
        ],
        scratch_shapes=[pltpu.VMEM((TN, N), jnp.float32)],
    )(xt, xf, wut, wvt)


def _sc_gather_rows(tab, ifl, G=16):
    B, N, D = tab.shape
    info = plsc.get_sparse_core_info()
    NC = info.num_cores
    NW = NC * info.num_subcores
    WPB = NW // B
    NP = N // WPB
    mesh = plsc.VectorSubcoreMesh(core_axis_name="c", subcore_axis_name="s")

    NG = NP // G

    @functools.partial(
        pl.kernel, mesh=mesh,
        out_type=jax.ShapeDtypeStruct((B, N * K, D), jnp.float32),
        scratch_types=[
            pltpu.VMEM((NP * K,), jnp.int32),
            pltpu.VMEM((G * K, D), jnp.float32),
            pltpu.VMEM((G * K, D), jnp.float32),
            pltpu.SemaphoreType.DMA,
            pltpu.SemaphoreType.DMA,
        ],
    )
    def k(t_hbm, i_hbm, o_hbm, idx_v, rows_a, rows_b, sem_a, sem_b):
        wid = lax.axis_index("s") * NC + lax.axis_index("c")
        b = wid // WPB
        p0 = (wid % WPB) * NP
        pltpu.sync_copy(i_hbm.at[b, pl.ds(p0 * K, NP * K)], idx_v)
        bufs = [(rows_a, sem_a), (rows_b, sem_b)]

        def start(g):
            rows, sem = bufs[g % 2]
            return pltpu.async_copy(
                t_hbm.at[b].at[idx_v.at[pl.ds(g * (G * K), G * K)]],
                rows, sem)

        cps = {0: start(0)}
        for g in range(NG):
            cps[g].wait()
            if g + 1 < NG:
                cps[g + 1] = start(g + 1)
            rows, _ = bufs[g % 2]
            pltpu.sync_copy(
                rows, o_hbm.at[b, pl.ds((p0 + g * G) * K, G * K), :])

    return k(tab, ifl)


def _conv_body(xj_ref, xt_ref, w_ref, h_ref, *, TN, C, CP2, OUT):
    xj = xj_ref[0][:, :C]
    xn = xt_ref[0]
    xnk = jnp.broadcast_to(xn[:, None, :], (TN, K, C)).reshape(TN * K, C)
    a = (xj - xnk).astype(jnp.bfloat16)
    bq = xnk.astype(jnp.bfloat16)
    parts = [a, bq]
    if CP2 > 2 * C:
        parts.append(jnp.zeros((TN * K, CP2 - 2 * C), jnp.bfloat16))
    feat = jnp.concatenate(parts, axis=1)
    y = lax.dot_general(feat, w_ref[...], (((1,), (0,)), ((), ())),
                        preferred_element_type=jnp.float32)
    y = y.reshape(TN, K, OUT)
    h = jnp.max(y, axis=1)
    h_ref[0] = jnp.where(h >= 0, h, SLOPE * h)


def _conv_tc(xj, xt, w16, TN=128):
    B, N, C = xt.shape
    CP2, OUT = w16.shape
    body = functools.partial(_conv_body, TN=TN, C=C, CP2=CP2, OUT=OUT)
    return pl.pallas_call(
        body,
        grid=(B, N // TN),
        in_specs=[
            pl.BlockSpec((1, TN * K, DP), lambda b, j: (b, j, 0)),
            pl.BlockSpec((1, TN, C), lambda b, j: (b, j, 0)),
            pl.BlockSpec((CP2, OUT), lambda b, j: (0, 0)),
        ],
        out_specs=pl.BlockSpec((1, TN, OUT), lambda b, j: (b, j, 0)),
        out_shape=jax.ShapeDtypeStruct((B, N, OUT), jnp.float32),
    )(xj, xt, w16)


def _sc_gather_max(ut, vfl, ifl, OUT, G=8):
    B, N, OUTP = ut.shape
    info = plsc.get_sparse_core_info()
    NC = info.num_cores
    NW = NC * info.num_subcores
    WPB = NW // B
    NP = N // WPB
    mesh = plsc.VectorSubcoreMesh(core_axis_name="c", subcore_axis_name="s")

    NG = NP // G

    @functools.partial(
        pl.kernel, mesh=mesh,
        out_type=jax.ShapeDtypeStruct((B, N * OUT), jnp.float32),
        scratch_types=[
            pltpu.VMEM((NP * K,), jnp.int32),
            pltpu.VMEM((G * K, OUTP), jnp.float32),
            pltpu.VMEM((G * K, OUTP), jnp.float32),
            pltpu.VMEM((G * OUT,), jnp.float32),
            pltpu.VMEM((G * OUT,), jnp.float32),
            pltpu.SemaphoreType.DMA,
            pltpu.SemaphoreType.DMA,
        ],
    )
    def k(u_hbm, v_hbm, i_hbm, h_hbm, idx_v, rows_a, rows_b, vg, hg,
          sem_a, sem_b):
        wid = lax.axis_index("s") * NC + lax.axis_index("c")
        b = wid // WPB
        p0 = (wid % WPB) * NP
        pltpu.sync_copy(i_hbm.at[b, pl.ds(p0 * K, NP * K)], idx_v)
        GK = G * K

        def gsrc(g):
            return u_hbm.at[b].at[idx_v.at[pl.ds(g * GK, GK)]]

        def compute(g, rows):
            pltpu.sync_copy(
                v_hbm.at[b, pl.ds((p0 + g * G) * OUT, G * OUT)], vg)

            def c_body(c, carry2):
                co = c * 16
                for p in range(G):
                    m = jnp.full((16,), -jnp.inf, jnp.float32)
                    for kk in range(K):
                        m = jnp.maximum(m, rows[p * K + kk, pl.ds(co, 16)])
                    t = m + vg[pl.ds(p * OUT + co, 16)]
                    hg[pl.ds(p * OUT + co, 16)] = (
                        jnp.where(t >= 0, t, SLOPE * t))
                return carry2

            lax.fori_loop(0, OUT // 16, c_body, 0)
            pltpu.sync_copy(
                hg, h_hbm.at[b, pl.ds((p0 + g * G) * OUT, G * OUT)])

        pltpu.async_copy(gsrc(0), rows_a, sem_a)

        def g_body(g2, carry):
            ga = 2 * g2
            gb = ga + 1
            pltpu.make_async_copy(gsrc(ga), rows_a, sem_a).wait()
            pltpu.async_copy(gsrc(gb), rows_b, sem_b)
            compute(ga, rows_a)
            pltpu.make_async_copy(gsrc(gb), rows_b, sem_b).wait()

            @pl.when(g2 < NG // 2 - 1)
            def _():
                pltpu.async_copy(gsrc(gb + 1), rows_a, sem_a)

            compute(gb, rows_b)
            return carry

        lax.fori_loop(0, NG // 2, g_body, 0)

    return k(ut, vfl, ifl)


def _final_body(h1_ref, h2_ref, h3_ref, h4_ref, w1_ref, w2_ref, w3_ref,
                w4_ref, out_ref, *, NT):
    j = pl.program_id(1)
    dn = (((1,), (0,)), ((), ()))
    g = lax.dot_general(w1_ref[...], h1_ref[0], dn,
                        preferred_element_type=jnp.float32)
    g += lax.dot_general(w2_ref[...], h2_ref[0], dn,
                         preferred_element_type=jnp.float32)
    g += lax.dot_general(w3_ref[...], h3_ref[0], dn,
                         preferred_element_type=jnp.float32)
    g += lax.dot_general(w4_ref[...], h4_ref[0], dn,
                         preferred_element_type=jnp.float32)
    part = jnp.max(g, axis=1)[None, None, :]

    @pl.when(j == 0)
    def _():
        out_ref[...] = jnp.full(out_ref.shape, -jnp.inf, jnp.float32)

    acc = jnp.maximum(out_ref[...], part)
    last = j == NT - 1
    out_ref[...] = jnp.where(last, jnp.where(acc >= 0, acc, SLOPE * acc), acc)


def _final(h1, h2, h3, h4, Wg, TND=512):
    B, _, N = h1.shape
    NT = N // TND
    O = Wg.shape[0]
    c1, c2, c3 = h1.shape[1], h2.shape[1], h3.shape[1]
    w1 = Wg[:, :c1]
    w2 = Wg[:, c1:c1 + c2]
    w3 = Wg[:, c1 + c2:c1 + c2 + c3]
    w4 = Wg[:, c1 + c2 + c3:]
    body = functools.partial(_final_body, NT=NT)

    def hspec(C):
        return pl.BlockSpec((1, C, TND), lambda b, j: (b, 0, j))

    def wspec(C):
        return pl.BlockSpec((O, C), lambda b, j: (0, 0))

    return pl.pallas_call(
        body,
        grid=(B, NT),
        in_specs=[hspec(c1), hspec(c2), hspec(c3), hspec(h4.shape[1]),
                  wspec(c1), wspec(c2), wspec(c3), wspec(w4.shape[1])],
        out_specs=pl.BlockSpec((1, 1, O), lambda b, j: (b, 0, 0)),
        out_shape=jax.ShapeDtypeStruct((B, 1, O), jnp.float32),
    )(h1, h2, h3, h4, w1, w2, w3, w4).reshape(B, O)


def _exact_block(xf_knn, xt_knn, xt, W):
    B, N, C = xt.shape
    OUT, C2 = W.shape
    CP2 = max(2 * C, 8)
    w16 = jnp.pad(W.astype(jnp.bfloat16).T, ((0, CP2 - C2), (0, 0)))
    idx = _knn_tc(xt_knn, xf_knn)
    tab = jnp.pad(xt, ((0, 0), (0, 0), (0, DP - C)))
    xj = _sc_gather_rows(tab, idx.reshape(B, N * K))
    return _conv_tc(xj, xt, w16)


def _fast_block(xf, xt, W):
    B, N, C = xt.shape
    OUT = W.shape[0]
    OUTP = max(OUT, 128)
    wa = W[:, :C].T
    wv = (W[:, C:] - W[:, :C]).T
    wut = jnp.pad(wa, ((0, 0), (0, OUTP - OUT)))
    idx, ut, vt = _knn_uv_tc(xt, xf, wut, wv)
    hfl = _sc_gather_max(ut, vt.reshape(B, N * OUT), idx.reshape(B, N * K),
                         OUT)
    return hfl.reshape(B, N, OUT)


def kernel(x, W1, W2, W3, W4, Wg):
    B, C0, N = x.shape
    xp = jnp.pad(x, ((0, 0), (0, 8 - C0), (0, 0)))
    xtp = jnp.transpose(xp, (0, 2, 1))
    h1t = _exact_block(xp, xtp, xtp[:, :, :C0], W1)
    h1 = jnp.transpose(h1t, (0, 2, 1))
    h2t = _exact_block(h1, h1t, h1t, W2)
    h2 = jnp.transpose(h2t, (0, 2, 1))
    h3t = _exact_block(h2, h2t, h2t, W3)
    h3 = jnp.transpose(h3t, (0, 2, 1))
    h4t = _fast_block(h3, h3t, W4)
    h4 = jnp.transpose(h4t, (0, 2, 1))
    return _final(h1, h2, h3, h4, Wg)
